# baseline (speedup 1.0000x reference)
import functools

import jax
import jax.numpy as jnp
from jax import lax
from jax.experimental import pallas as pl
from jax.experimental.pallas import tpu as pltpu
from jax.experimental.pallas import tpu_sc as plsc

NC = 2
NS = 16
NW = NC * NS
DP = 128
CHUNK = 128
GROUP = 2


@functools.partial(jax.jit, static_argnames=("total",))
def _sc_gather(idx2d, table_pad, total):
    bpw = total // NW
    nchunk = bpw // CHUNK
    ngroup = nchunk // GROUP
    gr = GROUP * CHUNK

    mesh = plsc.VectorSubcoreMesh(core_axis_name="c", subcore_axis_name="s")

    @functools.partial(
        pl.kernel,
        out_type=jax.ShapeDtypeStruct((total, DP), jnp.float32),
        mesh=mesh,
        scratch_types=[
            pltpu.VMEM((nchunk, CHUNK), jnp.int32),
            pltpu.VMEM((2, gr, DP), jnp.float32),
            pltpu.SemaphoreType.DMA,
            pltpu.SemaphoreType.DMA,
            pltpu.SemaphoreType.DMA,
            pltpu.SemaphoreType.DMA,
        ],
    )
    def gather_kernel(idx_hbm, table_hbm, out_hbm, idx_v, rows_v,
                      gsem0, gsem1, osem0, osem1):
        wid = lax.axis_index("s") * NC + lax.axis_index("c")
        base = wid * bpw
        gsem = (gsem0, gsem1)
        osem = (osem0, osem1)
        pltpu.sync_copy(idx_hbm.at[pl.ds(wid * nchunk, nchunk)], idx_v)

        def issue_gathers(g, s):
            for u in range(GROUP):
                pltpu.async_copy(
                    table_hbm.at[idx_v.at[g * GROUP + u]],
                    rows_v.at[s].at[pl.ds(u * CHUNK, CHUNK)],
                    gsem[s],
                )

        def drain_gathers(s):
            for u in range(GROUP):
                pltpu.make_async_copy(
                    table_hbm.at[idx_v.at[u]],
                    rows_v.at[s].at[pl.ds(u * CHUNK, CHUNK)],
                    gsem[s],
                ).wait()

        def drain_out(s):
            pltpu.make_async_copy(
                rows_v.at[s], out_hbm.at[pl.ds(base, gr)], osem[s],
            ).wait()

        issue_gathers(0, 0)

        @pl.loop(0, ngroup, step=2)
        def _(go):
            for s in range(2):
                g = go + s
                s2 = 1 - s
                drain_gathers(s)
                pltpu.async_copy(rows_v.at[s],
                                 out_hbm.at[pl.ds(base + g * gr, gr)],
                                 osem[s])
                if s == 0:
                    @pl.when(go > 0)
                    def _():
                        drain_out(s2)
                    issue_gathers(g + 1, s2)
                else:
                    drain_out(s2)

                    @pl.when(go + 2 < ngroup)
                    def _():
                        issue_gathers(g + 1, s2)

        drain_out(1)

    return gather_kernel(idx2d, table_pad)


def kernel(category_ids, embedding_weight):
    b, h = category_ids.shape
    total = b * h
    d = embedding_weight.shape[1]
    idx2d = category_ids.reshape(total // CHUNK, CHUNK).astype(jnp.int32)
    table_pad = jnp.pad(embedding_weight, ((0, 0), (0, DP - d)))
    out = _sc_gather(idx2d, table_pad, total)
    return out[:, :d].reshape(b, h, d)


# speedup vs baseline: 1.3029x; 1.3029x over previous
import functools

import jax
import jax.numpy as jnp
from jax import lax
from jax.experimental import pallas as pl
from jax.experimental.pallas import tpu as pltpu
from jax.experimental.pallas import tpu_sc as plsc

NC = 2
NS = 16
NW = NC * NS
DP = 128
CHUNK = 128
GROUP = 2


@functools.partial(jax.jit, static_argnames=("total",))
def _sc_gather(idx2d, table_pad, total):
    bpw = total // NW
    nchunk = bpw // CHUNK
    ngroup = nchunk // GROUP
    gr = GROUP * CHUNK

    mesh = plsc.VectorSubcoreMesh(core_axis_name="c", subcore_axis_name="s")

    @functools.partial(
        pl.kernel,
        out_type=jax.ShapeDtypeStruct((total, DP), jnp.float32),
        mesh=mesh,
        scratch_types=[
            pltpu.VMEM((nchunk, CHUNK), jnp.int32),
            pltpu.VMEM((2, gr, DP), jnp.float32),
            pltpu.SemaphoreType.DMA,
            pltpu.SemaphoreType.DMA,
            pltpu.SemaphoreType.DMA,
            pltpu.SemaphoreType.DMA,
        ],
    )
    def gather_kernel(idx_hbm, table_hbm, out_hbm, idx_v, rows_v,
                      gsem0, gsem1, osem0, osem1):
        wid = lax.axis_index("s") * NC + lax.axis_index("c")
        base = wid * bpw
        gsem = (gsem0, gsem1)
        osem = (osem0, osem1)
        pltpu.sync_copy(idx_hbm.at[pl.ds(wid * nchunk, nchunk)], idx_v)

        def issue_gathers(g, s):
            for u in range(GROUP):
                pltpu.async_copy(
                    table_hbm.at[idx_v.at[g * GROUP + u]],
                    rows_v.at[s].at[pl.ds(u * CHUNK, CHUNK)],
                    gsem[s],
                )

        def drain_gathers(s):
            for u in range(GROUP):
                pltpu.make_async_copy(
                    table_hbm.at[idx_v.at[u]],
                    rows_v.at[s].at[pl.ds(u * CHUNK, CHUNK)],
                    gsem[s],
                ).wait()

        def drain_out(s):
            pltpu.make_async_copy(
                rows_v.at[s], out_hbm.at[pl.ds(base, gr)], osem[s],
            ).wait()

        issue_gathers(0, 0)

        @pl.loop(0, ngroup, step=2)
        def _(go):
            for s in range(2):
                g = go + s
                s2 = 1 - s
                drain_gathers(s)
                pltpu.async_copy(rows_v.at[s],
                                 out_hbm.at[pl.ds(base + g * gr, gr)],
                                 osem[s])
                if s == 0:
                    @pl.when(go > 0)
                    def _():
                        drain_out(s2)
                    issue_gathers(g + 1, s2)
                else:
                    drain_out(s2)

                    @pl.when(go + 2 < ngroup)
                    def _():
                        issue_gathers(g + 1, s2)

        drain_out(1)

    return gather_kernel(idx2d, table_pad)


def _pad_transpose_kernel(x_ref, o_ref):
    x = x_ref[...]
    xt = x.T
    o_ref[...] = jnp.concatenate([xt, jnp.zeros_like(xt)], axis=1)


@jax.jit
def _prep_table(table_t):
    d, n = table_t.shape
    bc = 2048
    return pl.pallas_call(
        _pad_transpose_kernel,
        grid=(pl.cdiv(n, bc),),
        in_specs=[pl.BlockSpec((d, bc), lambda i: (0, i))],
        out_specs=pl.BlockSpec((bc, DP), lambda i: (i, 0)),
        out_shape=jax.ShapeDtypeStruct((n, DP), jnp.float32),
    )(table_t)


def _to_native_kernel(h, d, x_ref, o_ref):
    x = x_ref[...]
    bc = x.shape[0] // h
    x3 = x.reshape(bc, h, DP)
    for t in range(h):
        o_ref[t] = x3[:, t, :d].T


@functools.partial(jax.jit, static_argnames=("b", "h", "d"))
def _to_native(rows, b, h, d):
    bc = 128
    return pl.pallas_call(
        functools.partial(_to_native_kernel, h, d),
        grid=(b // bc,),
        in_specs=[pl.BlockSpec((bc * h, DP), lambda i: (i, 0))],
        out_specs=pl.BlockSpec((h, d, bc), lambda i: (0, 0, i)),
        out_shape=jax.ShapeDtypeStruct((h, d, b), jnp.float32),
    )(rows)


def kernel(category_ids, embedding_weight):
    b, h = category_ids.shape
    total = b * h
    d = embedding_weight.shape[1]
    idx2d = category_ids.reshape(total // CHUNK, CHUNK).astype(jnp.int32)
    table_pad = _prep_table(embedding_weight.T)
    rows = _sc_gather(idx2d, table_pad, total)
    out_native = _to_native(rows, b, h, d)
    return out_native.transpose(2, 0, 1)


# speedup vs baseline: 1.3031x; 1.0001x over previous
import functools

import jax
import jax.numpy as jnp
from jax import lax
from jax.experimental import pallas as pl
from jax.experimental.pallas import tpu as pltpu
from jax.experimental.pallas import tpu_sc as plsc

NC = 2
NS = 16
NW = NC * NS
DP = 128
CHUNK = 128
GROUP = 2


@functools.partial(jax.jit, static_argnames=("total",))
def _sc_gather(idx2d, table_pad, total):
    bpw = total // NW
    nchunk = bpw // CHUNK
    ngroup = nchunk // GROUP
    gr = GROUP * CHUNK

    mesh = plsc.VectorSubcoreMesh(core_axis_name="c", subcore_axis_name="s")

    @functools.partial(
        pl.kernel,
        out_type=jax.ShapeDtypeStruct((total, DP), jnp.float32),
        mesh=mesh,
        scratch_types=[
            pltpu.VMEM((nchunk, CHUNK), jnp.int32),
            pltpu.VMEM((2, gr, DP), jnp.float32),
            pltpu.SemaphoreType.DMA,
            pltpu.SemaphoreType.DMA,
            pltpu.SemaphoreType.DMA,
            pltpu.SemaphoreType.DMA,
        ],
        compiler_params=pltpu.CompilerParams(
            skip_device_barrier=True,
            disable_bounds_checks=True,
            disable_semaphore_checks=True,
        ),
    )
    def gather_kernel(idx_hbm, table_hbm, out_hbm, idx_v, rows_v,
                      gsem0, gsem1, osem0, osem1):
        wid = lax.axis_index("s") * NC + lax.axis_index("c")
        base = wid * bpw
        gsem = (gsem0, gsem1)
        osem = (osem0, osem1)
        pltpu.sync_copy(idx_hbm.at[pl.ds(wid * nchunk, nchunk)], idx_v)

        def issue_gathers(g, s):
            for u in range(GROUP):
                pltpu.async_copy(
                    table_hbm.at[idx_v.at[g * GROUP + u]],
                    rows_v.at[s].at[pl.ds(u * CHUNK, CHUNK)],
                    gsem[s],
                )

        def drain_gathers(s):
            for u in range(GROUP):
                pltpu.make_async_copy(
                    table_hbm.at[idx_v.at[u]],
                    rows_v.at[s].at[pl.ds(u * CHUNK, CHUNK)],
                    gsem[s],
                ).wait()

        def drain_out(s):
            pltpu.make_async_copy(
                rows_v.at[s], out_hbm.at[pl.ds(base, gr)], osem[s],
            ).wait()

        issue_gathers(0, 0)

        @pl.loop(0, ngroup, step=2)
        def _(go):
            for s in range(2):
                g = go + s
                s2 = 1 - s
                drain_gathers(s)
                pltpu.async_copy(rows_v.at[s],
                                 out_hbm.at[pl.ds(base + g * gr, gr)],
                                 osem[s])
                if s == 0:
                    @pl.when(go > 0)
                    def _():
                        drain_out(s2)
                    issue_gathers(g + 1, s2)
                else:
                    drain_out(s2)

                    @pl.when(go + 2 < ngroup)
                    def _():
                        issue_gathers(g + 1, s2)

        drain_out(1)

    return gather_kernel(idx2d, table_pad)


def _pad_transpose_kernel(x_ref, o_ref):
    x = x_ref[...]
    xt = x.T
    o_ref[...] = jnp.concatenate([xt, jnp.zeros_like(xt)], axis=1)


@jax.jit
def _prep_table(table_t):
    d, n = table_t.shape
    bc = 2048
    return pl.pallas_call(
        _pad_transpose_kernel,
        grid=(pl.cdiv(n, bc),),
        in_specs=[pl.BlockSpec((d, bc), lambda i: (0, i))],
        out_specs=pl.BlockSpec((bc, DP), lambda i: (i, 0)),
        out_shape=jax.ShapeDtypeStruct((n, DP), jnp.float32),
    )(table_t)


def _to_native_kernel(h, d, x_ref, o_ref):
    x = x_ref[...]
    bc = x.shape[0] // h
    x3 = x.reshape(bc, h, DP)
    for t in range(h):
        o_ref[t] = x3[:, t, :d].T


@functools.partial(jax.jit, static_argnames=("b", "h", "d"))
def _to_native(rows, b, h, d):
    bc = 128
    return pl.pallas_call(
        functools.partial(_to_native_kernel, h, d),
        grid=(b // bc,),
        in_specs=[pl.BlockSpec((bc * h, DP), lambda i: (i, 0))],
        out_specs=pl.BlockSpec((h, d, bc), lambda i: (0, 0, i)),
        out_shape=jax.ShapeDtypeStruct((h, d, b), jnp.float32),
    )(rows)


def kernel(category_ids, embedding_weight):
    b, h = category_ids.shape
    total = b * h
    d = embedding_weight.shape[1]
    idx2d = category_ids.reshape(total // CHUNK, CHUNK).astype(jnp.int32)
    table_pad = _prep_table(embedding_weight.T)
    rows = _sc_gather(idx2d, table_pad, total)
    out_native = _to_native(rows, b, h, d)
    return out_native.transpose(2, 0, 1)
